# Initial kernel scaffold; baseline (speedup 1.0000x reference)
import jax
import jax.numpy as jnp
from jax.experimental import pallas as pl


def kernel(all_segment_idx, all_segment_label, all_label_filter, all_scores, transition_score):
    raise NotImplementedError("write your pallas kernel here")


# speedup vs baseline: 72.9208x; 72.9208x over previous
import jax
import jax.numpy as jnp
from jax.experimental import pallas as pl
from jax.experimental.pallas import tpu as pltpu

BIG_NEG = -1e30
_L = 512
_CPAD = 128


def _dp_kernel(sc_ref, lab_ref, kv_ref, gold_ref, t2_ref, tp_ref, out_ref, p_scr):
    n = sc_ref.shape[1]
    lane = jax.lax.broadcasted_iota(jnp.int32, (1, _CPAD), 1)
    p_scr[pl.ds(0, 1), :] = jnp.full((1, _CPAD), BIG_NEG, jnp.float32)

    def body(i, carry):
        z, gsum, prev_lab, ngold, nkeep, pcur = carry
        sci = sc_ref[:, pl.ds(i, 1), :][0, 0, 0]
        labi = lab_ref[:, pl.ds(i, 1), :][0, 0, 0]
        ki = kv_ref[:, pl.ds(i, 1), :][0, 0, 0]
        gi = gold_ref[:, pl.ds(i, 1), :][0, 0, 0]

        row = p_scr[pl.ds(ki, 1), :]
        tvec = t2_ref[pl.ds(labi, 1), :]
        v = row + tvec
        m = jnp.max(v)
        lse = m + jnp.log(jnp.sum(jnp.exp(v - m)))
        alpha = sci + jnp.logaddexp(jnp.float32(0.0), lse)
        z = jnp.logaddexp(z, alpha)

        kept = sci > jnp.float32(-1e29)
        upd = jnp.logaddexp(pcur, jnp.where(lane == labi, alpha, BIG_NEG))
        pnew = jnp.where(kept, upd, pcur)
        p_scr[pl.ds(i + 1, 1), :] = pnew

        isg = gi > 0
        trow = tp_ref[pl.ds(prev_lab, 1), :]
        tpair = jnp.sum(jnp.where(lane == labi, trow, jnp.float32(0.0)))
        gsum = gsum + jnp.where(
            isg, sci + jnp.where(ngold > 0, tpair, jnp.float32(0.0)),
            jnp.float32(0.0))
        prev_lab = jnp.where(isg, labi, prev_lab)
        ngold = ngold + jnp.where(isg, 1, 0).astype(jnp.int32)
        nkeep = nkeep + jnp.where(kept, 1, 0).astype(jnp.int32)
        return z, gsum, prev_lab, ngold, nkeep, pnew

    init = (jnp.float32(BIG_NEG), jnp.float32(0.0), jnp.int32(0),
            jnp.int32(0), jnp.int32(0),
            jnp.full((1, _CPAD), BIG_NEG, jnp.float32))
    z, gsum, _, ngold, nkeep, _ = jax.lax.fori_loop(0, n, body, init)
    contrib = (ngold >= 1) & (nkeep != ngold)
    loss = jnp.where(contrib, z - gsum, jnp.float32(0.0))
    out_ref[...] = jnp.broadcast_to(loss, (1, 8, 128))


def _prep(seg_idx, seg_label, label_filter, scores):
    starts = seg_idx[:, 0]
    ends = seg_idx[:, 1]
    gold = seg_label > 0
    amax = jnp.full((_L,), -1, dtype=jnp.int32).at[starts].max(
        jnp.where(gold, ends, -1).astype(jnp.int32))
    cmax = jax.lax.cummax(amax, axis=0)
    overlap = cmax[ends] >= starts
    valid = (seg_label == 0) & (label_filter > 0) & overlap
    keep = gold | valid
    perm1 = jnp.argsort(starts, stable=True)
    en_key = jnp.where(keep, ends, jnp.iinfo(ends.dtype).max)
    perm2 = jnp.argsort(en_key[perm1], stable=True)
    order = perm1[perm2]
    st = starts[order]
    keep_s = keep[order]
    labs = label_filter[order].astype(jnp.int32)
    gold_s = gold[order]
    sc = jnp.where(keep_s, scores[order], BIG_NEG).astype(jnp.float32)
    cnt = jnp.zeros((_L,), jnp.int32).at[ends].add(keep.astype(jnp.int32))
    cum = jnp.cumsum(cnt)
    kv = jnp.where(st > 0, cum[jnp.maximum(st - 1, 0)], 0).astype(jnp.int32)
    return sc, labs, kv, gold_s.astype(jnp.int32)


def kernel(all_segment_idx, all_segment_label, all_label_filter, all_scores,
           transition_score):
    B, N = all_scores.shape
    C = transition_score.shape[0]
    sc, labs, kv, gold = jax.vmap(_prep)(
        all_segment_idx, all_segment_label, all_label_filter, all_scores)
    t2 = jnp.zeros((16, _CPAD), jnp.float32).at[:C, :C].set(
        transition_score.T.astype(jnp.float32))
    tp = jnp.zeros((16, _CPAD), jnp.float32).at[:C, :C].set(
        transition_score.astype(jnp.float32))
    out = pl.pallas_call(
        _dp_kernel,
        grid=(B,),
        in_specs=[
            pl.BlockSpec((1, N, 1), lambda b: (b, 0, 0)),
            pl.BlockSpec((1, N, 1), lambda b: (b, 0, 0)),
            pl.BlockSpec((1, N, 1), lambda b: (b, 0, 0)),
            pl.BlockSpec((1, N, 1), lambda b: (b, 0, 0)),
            pl.BlockSpec((16, _CPAD), lambda b: (0, 0)),
            pl.BlockSpec((16, _CPAD), lambda b: (0, 0)),
        ],
        out_specs=pl.BlockSpec((1, 8, 128), lambda b: (b, 0, 0)),
        out_shape=jax.ShapeDtypeStruct((B, 8, 128), jnp.float32),
        scratch_shapes=[pltpu.VMEM((N + 8, _CPAD), jnp.float32)],
    )(sc[..., None], labs[..., None], kv[..., None], gold[..., None], t2, tp)
    return out[:, 0, 0].sum()
